# baseline (speedup 1.0000x reference)
import functools

import jax
import jax.numpy as jnp
from jax import lax
from jax.experimental import pallas as pl
from jax.experimental.pallas import tpu as pltpu
from jax.experimental.pallas import tpu_sc as plsc

N_NODES = 10000
N_EDGES = 160000
N_TRIPLETS = 160000
HID = 128
EDGE_BINS = 80
TRI_BINS = 40
EMB = 64

RB = 512
NW = 32
GCH = 128
SCH = 64

NP = 10240
EP = 163840

_INTERP = False


def _ceil_to(x, m):
    return (x + m - 1) // m * m


def _row_mask(pid, rb, rows):
    rid = pid * rb + lax.broadcasted_iota(jnp.int32, (rb, 1), 0)
    return rid < rows


def _stats_block(om):
    c = om.shape[1]
    cs = jnp.sum(om, axis=0, keepdims=True)
    css = jnp.sum(om * om, axis=0, keepdims=True)
    return jnp.concatenate([cs, css, jnp.zeros((6, c), jnp.float32)], axis=0)


def _mm(x, ws, bs, affine=None, stats_rows=0):
    rpad, k = x.shape
    nout = len(ws)
    grid = rpad // RB
    outs = [jax.ShapeDtypeStruct((rpad, w.shape[1]), jnp.float32) for w in ws]
    if stats_rows:
        assert nout == 1
        outs.append(jax.ShapeDtypeStruct((8, ws[0].shape[1]), jnp.float32))

    def body(*refs):
        x_ref = refs[0]
        w_refs = refs[1:1 + nout]
        b_refs = refs[1 + nout:1 + 2 * nout]
        if affine is not None:
            s_ref, t_ref = refs[1 + 2 * nout:3 + 2 * nout]
            o_refs = refs[3 + 2 * nout:]
        else:
            o_refs = refs[1 + 2 * nout:]
        xb = x_ref[...]
        if affine is not None:
            a = xb * s_ref[...] + t_ref[...]
            xb = a * jax.nn.sigmoid(a)
        for j in range(nout):
            o = jnp.dot(xb, w_refs[j][...],
                        preferred_element_type=jnp.float32) + b_refs[j][...]
            o_refs[j][...] = o
            if stats_rows and j == 0:
                pid = pl.program_id(0)
                m = _row_mask(pid, RB, stats_rows)
                om = jnp.where(m, o, 0.0)
                st = _stats_block(om)
                sref = o_refs[nout]

                @pl.when(pid == 0)
                def _():
                    sref[...] = st

                @pl.when(pid != 0)
                def _():
                    sref[...] = sref[...] + st

    in_specs = [pl.BlockSpec((RB, k), lambda i: (i, 0))]
    in_specs += [pl.BlockSpec(w.shape, lambda i: (0, 0)) for w in ws]
    in_specs += [pl.BlockSpec((1, w.shape[1]), lambda i: (0, 0)) for w in ws]
    args = [x] + list(ws) + [b.reshape(1, -1) for b in bs]
    if affine is not None:
        s, t = affine
        in_specs += [pl.BlockSpec((1, k), lambda i: (0, 0))] * 2
        args += [s.reshape(1, -1), t.reshape(1, -1)]
    out_specs = [pl.BlockSpec((RB, w.shape[1]), lambda i: (i, 0)) for w in ws]
    if stats_rows:
        out_specs.append(pl.BlockSpec((8, ws[0].shape[1]), lambda i: (0, 0)))

    res = pl.pallas_call(
        body, grid=(grid,), in_specs=in_specs, out_specs=out_specs,
        out_shape=outs, interpret=_INTERP)(*args)
    return list(res) if isinstance(res, (list, tuple)) else [res]


def _rbf_mm(v, w, b, vmin, vmax, bins, norm3, stats_rows):
    rpad = v.shape[0]
    c = w.shape[1]
    grid = rpad // RB
    gamma = 1.0 / ((vmax - vmin) / (bins - 1))
    step = (vmax - vmin) / (bins - 1)

    def body(v_ref, w_ref, b_ref, o_ref, s_ref):
        vb = v_ref[...]
        if norm3:
            d = jnp.sqrt(vb[:, 0:1] ** 2 + vb[:, 1:2] ** 2 + vb[:, 2:3] ** 2)
        else:
            d = vb[:, 0:1]
        centers = vmin + step * lax.broadcasted_iota(
            jnp.int32, (1, bins), 1).astype(jnp.float32)
        phi = jnp.exp(-gamma * (d - centers) ** 2)
        o = jnp.dot(phi, w_ref[...], preferred_element_type=jnp.float32) + b_ref[...]
        o_ref[...] = o
        pid = pl.program_id(0)
        m = _row_mask(pid, RB, stats_rows)
        om = jnp.where(m, o, 0.0)
        st = _stats_block(om)

        @pl.when(pid == 0)
        def _():
            s_ref[...] = st

        @pl.when(pid != 0)
        def _():
            s_ref[...] = s_ref[...] + st

    return pl.pallas_call(
        body, grid=(grid,),
        in_specs=[pl.BlockSpec((RB, 16), lambda i: (i, 0)),
                  pl.BlockSpec(w.shape, lambda i: (0, 0)),
                  pl.BlockSpec((1, c), lambda i: (0, 0))],
        out_specs=[pl.BlockSpec((RB, c), lambda i: (i, 0)),
                   pl.BlockSpec((8, c), lambda i: (0, 0))],
        out_shape=[jax.ShapeDtypeStruct((rpad, c), jnp.float32),
                   jax.ShapeDtypeStruct((8, c), jnp.float32)],
        interpret=_INTERP)(v, w.astype(jnp.float32), b.reshape(1, -1))


def _combine(g1, g2, ge, rows):
    rpad = g1.shape[0]
    grid = rpad // RB

    def body(g1_ref, g2_ref, ge_ref, us_ref, m_ref, s_ref):
        g1b = g1_ref[...]
        m = g1b[:, :HID] + g2_ref[...] + ge_ref[...]
        sig = jax.nn.sigmoid(m)
        u = g1b[:, HID:] * sig
        us_ref[...] = jnp.concatenate([u, sig], axis=1)
        m_ref[...] = m
        pid = pl.program_id(0)
        msk = _row_mask(pid, RB, rows)
        mm_ = jnp.where(msk, m, 0.0)
        st = _stats_block(mm_)

        @pl.when(pid == 0)
        def _():
            s_ref[...] = st

        @pl.when(pid != 0)
        def _():
            s_ref[...] = s_ref[...] + st

    return pl.pallas_call(
        body, grid=(grid,),
        in_specs=[pl.BlockSpec((RB, 2 * HID), lambda i: (i, 0)),
                  pl.BlockSpec((RB, HID), lambda i: (i, 0)),
                  pl.BlockSpec((RB, HID), lambda i: (i, 0))],
        out_specs=[pl.BlockSpec((RB, 2 * HID), lambda i: (i, 0)),
                   pl.BlockSpec((RB, HID), lambda i: (i, 0)),
                   pl.BlockSpec((8, HID), lambda i: (0, 0))],
        out_shape=[jax.ShapeDtypeStruct((rpad, 2 * HID), jnp.float32),
                   jax.ShapeDtypeStruct((rpad, HID), jnp.float32),
                   jax.ShapeDtypeStruct((8, HID), jnp.float32)],
        interpret=_INTERP)(g1, g2, ge)


def _node_combine(t3, s2, rows):
    rpad = t3.shape[0]
    grid = rpad // RB

    def body(t3_ref, s2_ref, x_ref, st_ref):
        sb = s2_ref[...]
        x = t3_ref[...] + sb[:, :HID] / (sb[:, HID:] + 1e-6)
        x_ref[...] = x
        pid = pl.program_id(0)
        msk = _row_mask(pid, RB, rows)
        xm = jnp.where(msk, x, 0.0)
        st = _stats_block(xm)

        @pl.when(pid == 0)
        def _():
            st_ref[...] = st

        @pl.when(pid != 0)
        def _():
            st_ref[...] = st_ref[...] + st

    return pl.pallas_call(
        body, grid=(grid,),
        in_specs=[pl.BlockSpec((RB, HID), lambda i: (i, 0)),
                  pl.BlockSpec((RB, 2 * HID), lambda i: (i, 0))],
        out_specs=[pl.BlockSpec((RB, HID), lambda i: (i, 0)),
                   pl.BlockSpec((8, HID), lambda i: (0, 0))],
        out_shape=[jax.ShapeDtypeStruct((rpad, HID), jnp.float32),
                   jax.ShapeDtypeStruct((8, HID), jnp.float32)],
        interpret=_INTERP)(t3, s2)


def _apply(x, s, t, resid=None, stats_rows=0):
    rpad, c = x.shape
    grid = rpad // RB
    outs = [jax.ShapeDtypeStruct((rpad, c), jnp.float32)]
    if stats_rows:
        outs.append(jax.ShapeDtypeStruct((8, c), jnp.float32))

    def body(*refs):
        if resid is not None:
            x_ref, s_ref, t_ref, r_ref = refs[:4]
            o_refs = refs[4:]
        else:
            x_ref, s_ref, t_ref = refs[:3]
            o_refs = refs[3:]
        a = x_ref[...] * s_ref[...] + t_ref[...]
        o = a * jax.nn.sigmoid(a)
        if resid is not None:
            o = r_ref[...] + o
        o_refs[0][...] = o
        if stats_rows:
            pid = pl.program_id(0)
            msk = _row_mask(pid, RB, stats_rows)
            om = jnp.where(msk, o, 0.0)
            st = _stats_block(om)

            @pl.when(pid == 0)
            def _():
                o_refs[1][...] = st

            @pl.when(pid != 0)
            def _():
                o_refs[1][...] = o_refs[1][...] + st

    in_specs = [pl.BlockSpec((RB, c), lambda i: (i, 0)),
                pl.BlockSpec((1, c), lambda i: (0, 0)),
                pl.BlockSpec((1, c), lambda i: (0, 0))]
    args = [x, s.reshape(1, -1), t.reshape(1, -1)]
    if resid is not None:
        in_specs.append(pl.BlockSpec((RB, c), lambda i: (i, 0)))
        args.append(resid)
    out_specs = [pl.BlockSpec((RB, c), lambda i: (i, 0))]
    if stats_rows:
        out_specs.append(pl.BlockSpec((8, c), lambda i: (0, 0)))
    res = pl.pallas_call(
        body, grid=(grid,), in_specs=in_specs, out_specs=out_specs,
        out_shape=outs, interpret=_INTERP)(*args)
    return list(res) if isinstance(res, (list, tuple)) else [res]


def _sc_gather(table, idx):
    v, d = table.shape
    b = idx.shape[0]
    bpw = b // NW
    nch = bpw // GCH
    mesh = plsc.VectorSubcoreMesh(core_axis_name="c", subcore_axis_name="s")

    @functools.partial(
        pl.kernel,
        out_type=jax.ShapeDtypeStruct((b, d), jnp.float32),
        mesh=mesh,
        scratch_types=[pltpu.VMEM((GCH,), jnp.int32),
                       pltpu.VMEM((GCH, d), jnp.float32),
                       pltpu.SemaphoreType.DMA],
        interpret=_INTERP)
    def k(table_hbm, idx_hbm, out_hbm, idx_v, rows_v, sem):
        wid = lax.axis_index("s") * 2 + lax.axis_index("c")
        base = wid * bpw

        def body(i, _):
            off = base + i * GCH
            pltpu.sync_copy(idx_hbm.at[pl.ds(off, GCH)], idx_v)
            pltpu.async_copy(table_hbm.at[idx_v], rows_v, sem).wait()
            pltpu.sync_copy(rows_v, out_hbm.at[pl.ds(off, GCH)])
            return 0

        lax.fori_loop(0, nch, body, 0)

    return k(table, idx)


def _sc_segsum(vals, seg, bnd, nseg_pad):
    d = vals.shape[1]
    spw = nseg_pad // NW
    nch = spw // SCH
    nb8 = bnd.shape[0]
    mesh = plsc.VectorSubcoreMesh(core_axis_name="c", subcore_axis_name="s")

    @functools.partial(
        pl.kernel,
        out_type=jax.ShapeDtypeStruct((nseg_pad, d), jnp.float32),
        mesh=mesh,
        scratch_types=[pltpu.VMEM((GCH, d), jnp.float32),
                       pltpu.VMEM((GCH,), jnp.int32),
                       pltpu.VMEM((SCH, d), jnp.float32),
                       pltpu.VMEM((nb8 + 16,), jnp.int32)],
        interpret=_INTERP)
    def k(vals_hbm, seg_hbm, bnd_hbm, out_hbm, vbuf, sbuf, acc, bndv):
        wid = lax.axis_index("s") * 2 + lax.axis_index("c")
        pltpu.sync_copy(bnd_hbm, bndv.at[pl.ds(0, nb8)])
        zero16 = jnp.zeros((16,), jnp.float32)

        def chunk_body(c, _):
            chunkbase = pl.multiple_of(wid * spw + c * SCH, SCH)

            def zbody(srow, _):
                for j in range(d // 16):
                    acc[srow, pl.ds(j * 16, 16)] = zero16
                return 0

            lax.fori_loop(0, SCH, zbody, 0)

            bv = bndv[pl.ds(wid * nch + c, 16)]
            e0 = bv[0]
            e1 = bv[1]
            a0 = e0 & ~(GCH - 1)
            nk = (e1 - a0 + GCH - 1) // GCH

            def kbody(kk, _):
                kb = pl.multiple_of(a0 + kk * GCH, GCH)
                pltpu.sync_copy(vals_hbm.at[pl.ds(kb, GCH)], vbuf)
                pltpu.sync_copy(seg_hbm.at[pl.ds(kb, GCH)], sbuf)

                def ebody(i16, _):
                    sv = sbuf[pl.ds(i16 * 16, 16)] - chunkbase
                    for jj in range(16):
                        ls = sv[jj]

                        @pl.when((ls >= 0) & (ls < SCH))
                        def _():
                            for j in range(d // 16):
                                sl = pl.ds(j * 16, 16)
                                plsc.addupdate(acc.at[ls, sl],
                                               vbuf[i16 * 16 + jj, sl])

                    return 0

                lax.fori_loop(0, GCH // 16, ebody, 0)
                return 0

            lax.fori_loop(0, nk, kbody, 0)
            pltpu.sync_copy(acc, out_hbm.at[pl.ds(chunkbase, SCH)])
            return 0

        lax.fori_loop(0, nch, chunk_body, 0)

    return k(vals, seg, bnd)


def _bn_affine(stats, rows, g, be):
    cs, css = stats[0], stats[1]
    mu = cs / rows
    var = css / rows - mu * mu
    s = g * lax.rsqrt(var + 1e-5)
    t = be - mu * s
    return s, t


def _mlp_affine(p, stats, rows):
    return _bn_affine(stats, rows, p["g"], p["be"])


def _pad_rows(x, rpad):
    return jnp.pad(x, ((0, rpad - x.shape[0]),) + ((0, 0),) * (x.ndim - 1))


def _seg_tables(seg_sorted, real, nseg_pad):
    segp = jnp.concatenate([
        seg_sorted.astype(jnp.int32),
        jnp.full((EP - real,), nseg_pad, jnp.int32)])
    marks = jnp.arange(0, nseg_pad + 1, SCH, dtype=jnp.int32)
    bnd = jnp.searchsorted(segp, marks, side="left").astype(jnp.int32)
    nb8 = _ceil_to(bnd.shape[0], 8)
    bnd = jnp.pad(bnd, (0, nb8 - bnd.shape[0]), mode="edge")
    return segp, bnd


def _eggc(p, src_i, dst_i, segp, bnd, nseg_pad, nrows_h, nrows_e, h, e):
    w1 = jnp.concatenate([p["src_gate"]["W"], p["dst_update"]["W"]], axis=1)
    b1 = jnp.concatenate([p["src_gate"]["b"], p["dst_update"]["b"]])
    t1, t2, t3 = _mm(h, [w1, p["dst_gate"]["W"], p["src_update"]["W"]],
                     [b1, p["dst_gate"]["b"], p["src_update"]["b"]])
    ge = _mm(e, [p["edge_gate"]["W"]], [p["edge_gate"]["b"]])[0]
    g1 = _sc_gather(t1, src_i)
    g2 = _sc_gather(t2, dst_i)
    usig, m, m_stats = _combine(g1, g2, ge, nrows_e)
    s2 = _sc_segsum(usig, segp, bnd, nseg_pad)
    x, x_stats = _node_combine(t3, s2[:t3.shape[0]], nrows_h)
    sx, tx = _bn_affine(x_stats, nrows_h, p["bn_n_g"], p["bn_n_b"])
    sm, tm = _bn_affine(m_stats, nrows_e, p["bn_e_g"], p["bn_e_b"])
    h_new, h_stats = _apply(x, sx, tx, resid=h, stats_rows=nrows_h)
    e_new = _apply(m, sm, tm, resid=e)[0]
    return h_new, e_new, m, h_stats


def kernel(atom_features, r, angle_h, params, edge_index, lg_edge_index):
    src = edge_index[0].astype(jnp.int32)
    dst = edge_index[1].astype(jnp.int32)
    lsrc = lg_edge_index[0].astype(jnp.int32)
    ldst = lg_edge_index[1].astype(jnp.int32)

    perm_e = jnp.argsort(dst)
    src_s = src[perm_e]
    dst_s = dst[perm_e]
    inv_e = jnp.zeros((N_EDGES,), jnp.int32).at[perm_e].set(
        jnp.arange(N_EDGES, dtype=jnp.int32))
    lsrc2 = inv_e[lsrc]
    ldst2 = inv_e[ldst]
    perm_t = jnp.argsort(ldst2)
    lsrc_s = lsrc2[perm_t]
    ldst_s = ldst2[perm_t]

    pad_i = lambda ix: jnp.pad(ix, (0, EP - ix.shape[0]))
    src_i = pad_i(src_s)
    dst_i = pad_i(dst_s)
    lsrc_i = pad_i(lsrc_s)
    ldst_i = pad_i(ldst_s)

    segp_n, bnd_n = _seg_tables(dst_s, N_EDGES, NP)
    segp_e, bnd_e = _seg_tables(ldst_s, N_TRIPLETS, EP)

    r_s = _pad_rows(jnp.pad(r[perm_e], ((0, 0), (0, 13))), EP)
    ah_s = _pad_rows(jnp.pad(angle_h[perm_t][:, None], ((0, 0), (0, 15))), EP)

    pa = params["atom_emb"]
    o0, st0 = _mm(_pad_rows(atom_features, NP), [pa["lin"]["W"]],
                  [pa["lin"]["b"]], stats_rows=N_NODES)
    x = _apply(o0, *_mlp_affine(pa, st0, N_NODES))[0]

    pe1, pe2 = params["edge_emb1"], params["edge_emb2"]
    oy1, sty1 = _rbf_mm(r_s, pe1["lin"]["W"], pe1["lin"]["b"],
                        0.0, 8.0, EDGE_BINS, True, N_EDGES)
    oy2, sty2 = _mm(oy1, [pe2["lin"]["W"]], [pe2["lin"]["b"]],
                    affine=_mlp_affine(pe1, sty1, N_EDGES),
                    stats_rows=N_EDGES)
    y = _apply(oy2, *_mlp_affine(pe2, sty2, N_EDGES))[0]

    pz1, pz2 = params["ang_emb1"], params["ang_emb2"]
    oz1, stz1 = _rbf_mm(ah_s, pz1["lin"]["W"], pz1["lin"]["b"],
                        -1.0, 1.0, TRI_BINS, False, N_TRIPLETS)
    oz2, stz2 = _mm(oz1, [pz2["lin"]["W"]], [pz2["lin"]["b"]],
                    affine=_mlp_affine(pz1, stz1, N_TRIPLETS),
                    stats_rows=N_TRIPLETS)
    z = _apply(oz2, *_mlp_affine(pz2, stz2, N_TRIPLETS))[0]

    h_stats = None
    for lp in params["alignn"]:
        x, y, m, _ = _eggc(lp["node"], src_i, dst_i, segp_n, bnd_n, NP,
                           N_NODES, N_EDGES, x, y)
        y, z, _, _ = _eggc(lp["edge"], lsrc_i, ldst_i, segp_e, bnd_e, EP,
                           N_EDGES, N_TRIPLETS, m, z)
    for gp in params["gcn"]:
        x, y, _, h_stats = _eggc(gp, src_i, dst_i, segp_n, bnd_n, NP,
                                 N_NODES, N_EDGES, x, y)

    hpool = h_stats[0] / N_NODES
    out = hpool @ params["fc"]["W"] + params["fc"]["b"]
    return jnp.squeeze(out)
